# Initial kernel scaffold; baseline (speedup 1.0000x reference)
import jax
import jax.numpy as jnp
from jax.experimental import pallas as pl


def kernel(x, edge_index, edge_attr, W_in, b_in, W1_0, b1_0, W2_0, b2_0, Wr_0, br_0, gamma_0, beta_0, W1_1, b1_1, W2_1, b2_1, Wr_1, br_1, gamma_1, beta_1, W1_2, b1_2, W2_2, b2_2, Wr_2, br_2, gamma_2, beta_2):
    raise NotImplementedError("write your pallas kernel here")


# speedup vs baseline: 1.7780x; 1.7780x over previous
import functools

import jax
import jax.numpy as jnp
from jax import lax
from jax.experimental import pallas as pl
from jax.experimental.pallas import tpu as pltpu
from jax.experimental.pallas import tpu_sc as plsc

N = 10000
E = 160000
H = 16
EPS = 1e-5

NC = 2
NS = 16
NW = NC * NS
CHUNK = E // NW
NROWS = N // NS

_mesh = plsc.VectorSubcoreMesh(core_axis_name="c", subcore_axis_name="s")
_sc_params = pltpu.CompilerParams(needs_layout_passes=False)


NGRP = CHUNK // 16 + 1
CPAD = NGRP * 16


def _gather_half(idx_v, htab, rows_v):
    lanes = jnp.arange(16, dtype=jnp.int32)

    def body(k, carry):
        sv = idx_v[pl.ds(k * 16, 16)]
        base = sv * 8
        eids = k * 16 + lanes
        for w in range(8):
            vals = plsc.load_gather(htab, [base + w])
            plsc.store_scatter(rows_v, [eids * 8 + w], vals)
        return carry

    lax.fori_loop(0, NGRP, body, 0)


@functools.partial(
    pl.kernel,
    mesh=_mesh,
    out_type=[jax.ShapeDtypeStruct((E * 8,), jnp.float32),
              jax.ShapeDtypeStruct((E * 8,), jnp.float32)],
    compiler_params=_sc_params,
    scratch_types=[
        pltpu.VMEM((CPAD + 16,), jnp.int32),
        pltpu.VMEM((N * 8,), jnp.float32),
        pltpu.VMEM((CPAD * 8,), jnp.float32),
    ],
)
def _sc_gather(ha_hbm, hb_hbm, src_hbm, outa_hbm, outb_hbm, idx_v, htab, rows_v):
    wid = lax.axis_index("s") * NC + lax.axis_index("c")
    base = wid * CHUNK
    pltpu.sync_copy(src_hbm.at[pl.ds(base, CHUNK)], idx_v.at[pl.ds(0, CHUNK)])
    idx_v[pl.ds(CHUNK, 16)] = jnp.zeros((16,), jnp.int32)
    pltpu.sync_copy(ha_hbm, htab)
    _gather_half(idx_v, htab, rows_v)
    pltpu.sync_copy(rows_v.at[pl.ds(0, CHUNK * 8)],
                    outa_hbm.at[pl.ds(base * 8, CHUNK * 8)])
    pltpu.sync_copy(hb_hbm, htab)
    _gather_half(idx_v, htab, rows_v)
    pltpu.sync_copy(rows_v.at[pl.ds(0, CHUNK * 8)],
                    outb_hbm.at[pl.ds(base * 8, CHUNK * 8)])


APAD = 640
NPAIR = CHUNK // 2


@functools.partial(
    pl.kernel,
    mesh=_mesh,
    out_type=[jax.ShapeDtypeStruct((NC, APAD, 128), jnp.float32),
              jax.ShapeDtypeStruct((NC, APAD, 128), jnp.float32)],
    compiler_params=_sc_params,
    scratch_types=[
        pltpu.VMEM((CHUNK,), jnp.int32),
        pltpu.VMEM((CHUNK * 4,), jnp.float32),
        pltpu.VMEM((APAD, 128), jnp.float32),
        pltpu.VMEM((APAD,), jnp.int32),
        pltpu.VMEM_SHARED((APAD, 128), jnp.float32),
    ],
)
def _sc_scatter(msga_hbm, msgb_hbm, dst_hbm, zero_hbm, outa_hbm, outb_hbm,
                idx_v, vals_v, acc_v, iota_v, accum_sh):
    c = lax.axis_index("c")
    s = lax.axis_index("s")
    wid = s * NC + c
    base = wid * CHUNK
    lanes = jnp.arange(16, dtype=jnp.int32)
    lo = lanes < 8
    hi = lanes >= 8

    pltpu.sync_copy(dst_hbm.at[pl.ds(base, CHUNK)], idx_v)

    def iota_fill(r, carry):
        iota_v[pl.ds(r * 16, 16)] = r * 16 + lanes
        return carry

    lax.fori_loop(0, APAD // 16, iota_fill, 0)

    for msg_hbm, out_hbm in ((msga_hbm, outa_hbm), (msgb_hbm, outb_hbm)):
        pltpu.sync_copy(zero_hbm, acc_v)
        pltpu.sync_copy(zero_hbm.at[pl.ds(s * 40, 40)],
                        accum_sh.at[pl.ds(s * 40, 40)])
        for sub in range(2):
            pltpu.sync_copy(
                msg_hbm.at[pl.ds(base * 8 + sub * (CHUNK * 4), CHUNK * 4)],
                vals_v)
            e0 = sub * (CHUNK // 2)

            def pair(k, carry):
                dpair = plsc.load_gather(idx_v, [e0 + 2 * k + (lanes >> 3)])
                a = dpair * 8 + (lanes & 7)
                row = a >> 7
                col = a & 127
                vals = vals_v[pl.ds(k * 16, 16)]
                plsc.addupdate_scatter(acc_v, [row, col], vals, mask=lo)
                plsc.addupdate_scatter(acc_v, [row, col], vals, mask=hi)
                return carry

            lax.fori_loop(0, NPAIR // 2, pair, 0)
        plsc.subcore_barrier()
        pltpu.sync_copy(acc_v, accum_sh.at[iota_v], add=True)
        plsc.subcore_barrier()
        pltpu.sync_copy(accum_sh.at[pl.ds(s * 40, 40)],
                        out_hbm.at[c, pl.ds(s * 40, 40)])
        plsc.subcore_barrier()


def _mlp_body(x_ref, w_ref, b_ref, o_ref):
    y = jnp.dot(x_ref[...], w_ref[...], preferred_element_type=jnp.float32)
    o_ref[...] = jnp.maximum(y + b_ref[...], 0.0)


def _tc_input_mlp(x, W_in, b_in):
    BLK = 2000
    return pl.pallas_call(
        _mlp_body,
        grid=(N // BLK,),
        in_specs=[
            pl.BlockSpec((BLK, 128), lambda i: (i, 0)),
            pl.BlockSpec((128, H), lambda i: (0, 0)),
            pl.BlockSpec((1, H), lambda i: (0, 0)),
        ],
        out_specs=pl.BlockSpec((BLK, H), lambda i: (i, 0)),
        out_shape=jax.ShapeDtypeStruct((N, H), jnp.float32),
    )(x, W_in, b_in.reshape(1, H))


def _msg_body(ea_ref, ga_ref, gb_ref, w1r_ref, b1r_ref, ms_ref, f_ref, b2_ref,
              o_ref):
    ea = ea_ref[...]
    g = jnp.concatenate([ga_ref[...], gb_ref[...]], axis=1)
    tb = jnp.dot(ea, w1r_ref[...], preferred_element_type=jnp.float32)
    tb = jnp.maximum(tb + b1r_ref[...], 0.0)
    u = jnp.dot(g, ms_ref[...], preferred_element_type=jnp.float32)
    msg = jnp.dot(tb * u, f_ref[...], preferred_element_type=jnp.float32)
    msg = msg + jnp.dot(g, b2_ref[...], preferred_element_type=jnp.float32)
    o_ref[...] = msg


def _tc_msg(edge_attr, ga, gb, W1R, b1R, Mstack, F, B2r):
    BLK = 4000
    return pl.pallas_call(
        _msg_body,
        grid=(E // BLK,),
        in_specs=[
            pl.BlockSpec((BLK, 4), lambda i: (i, 0)),
            pl.BlockSpec((BLK, 8), lambda i: (i, 0)),
            pl.BlockSpec((BLK, 8), lambda i: (i, 0)),
            pl.BlockSpec((4, 128), lambda i: (0, 0)),
            pl.BlockSpec((1, 128), lambda i: (0, 0)),
            pl.BlockSpec((H, 128), lambda i: (0, 0)),
            pl.BlockSpec((128, H), lambda i: (0, 0)),
            pl.BlockSpec((H, H), lambda i: (0, 0)),
        ],
        out_specs=pl.BlockSpec((BLK, H), lambda i: (i, 0)),
        out_shape=jax.ShapeDtypeStruct((E, H), jnp.float32),
    )(edge_attr, ga, gb, W1R, b1R, Mstack, F, B2r)


def _node_body(h_ref, pa0_ref, pa1_ref, pb0_ref, pb1_ref, da0_ref, da1_ref,
               wr_ref, br_ref, gm_ref, bt_ref, o_ref):
    h = h_ref[...]
    dh = jnp.maximum(da0_ref[...] + da1_ref[...], 1.0)
    deg = jnp.concatenate([dh, dh], axis=1)
    agg = jnp.concatenate([pa0_ref[...] + pa1_ref[...],
                           pb0_ref[...] + pb1_ref[...]], axis=1) / deg
    u = jnp.dot(h, wr_ref[...], preferred_element_type=jnp.float32)
    u = u + br_ref[...] + agg
    u = u * gm_ref[...] + bt_ref[...]
    o_ref[...] = jnp.maximum(u, 0.0) + h


def _tc_node(h, pa0, pa1, pb0, pb1, da0, da1, Wr, br, gm, bt):
    BLK = 2000
    full = lambda r, c: pl.BlockSpec((r, c), lambda i: (0, 0))
    rowh = lambda: pl.BlockSpec((BLK, 8), lambda i: (i, 0))
    row = lambda: pl.BlockSpec((BLK, H), lambda i: (i, 0))
    return pl.pallas_call(
        _node_body,
        grid=(N // BLK,),
        in_specs=[row(), rowh(), rowh(), rowh(), rowh(), rowh(), rowh(),
                  full(H, H), full(1, H), full(1, H), full(1, H)],
        out_specs=row(),
        out_shape=jax.ShapeDtypeStruct((N, H), jnp.float32),
    )(h, pa0, pa1, pb0, pb1, da0, da1, Wr, br, gm, bt)


def kernel(x, edge_index, edge_attr, W_in, b_in,
           W1_0, b1_0, W2_0, b2_0, Wr_0, br_0, gamma_0, beta_0,
           W1_1, b1_1, W2_1, b2_1, Wr_1, br_1, gamma_1, beta_1,
           W1_2, b1_2, W2_2, b2_2, Wr_2, br_2, gamma_2, beta_2):
    f32 = jnp.float32
    src = edge_index[0]
    dst = edge_index[1]
    zeros_p = jnp.zeros((APAD, 128), f32)
    ones_e = jnp.ones((E * 8,), f32)
    rs = 1.0 / jnp.sqrt(jnp.asarray(1.0 + EPS, f32))
    F = jnp.tile(jnp.eye(H, dtype=f32), (8, 1))

    def unpack(p):
        return p.reshape(APAD * 128)[:N * 8].reshape(N, 8)

    h = _tc_input_mlp(x, W_in, b_in)

    dega, _ = _sc_scatter(ones_e, ones_e, dst, zeros_p)
    da0, da1 = unpack(dega[0]), unpack(dega[1])

    layers = [
        (W1_0, b1_0, W2_0, b2_0, Wr_0, br_0, gamma_0, beta_0),
        (W1_1, b1_1, W2_1, b2_1, Wr_1, br_1, gamma_1, beta_1),
        (W1_2, b1_2, W2_2, b2_2, Wr_2, br_2, gamma_2, beta_2),
    ]
    for (W1, b1, W2, b2, Wr, br, gm, bt) in layers:
        W1R = jnp.repeat(W1, H, axis=1)
        b1R = jnp.repeat(b1, H).reshape(1, 8 * H)
        Mstack = W2.reshape(8, H, H).transpose(1, 0, 2).reshape(H, 8 * H)
        B2r = b2.reshape(H, H)
        ha = h[:, :8].reshape(N * 8)
        hb = h[:, 8:].reshape(N * 8)
        ga, gb = _sc_gather(ha, hb, src)
        msg = _tc_msg(edge_attr, ga.reshape(E, 8), gb.reshape(E, 8),
                      W1R, b1R, Mstack, F, B2r)
        msga = msg[:, :8].reshape(E * 8)
        msgb = msg[:, 8:].reshape(E * 8)
        pa, pb = _sc_scatter(msga, msgb, dst, zeros_p)
        h = _tc_node(h, unpack(pa[0]), unpack(pa[1]),
                     unpack(pb[0]), unpack(pb[1]), da0, da1,
                     Wr, br.reshape(1, H), (gm * rs).reshape(1, H),
                     bt.reshape(1, H))
    return h


# speedup vs baseline: 1.9377x; 1.0899x over previous
import functools

import jax
import jax.numpy as jnp
from jax import lax
from jax.experimental import pallas as pl
from jax.experimental.pallas import tpu as pltpu
from jax.experimental.pallas import tpu_sc as plsc

N = 10000
E = 160000
H = 16
EPS = 1e-5

NC = 2
NS = 16
NW = NC * NS
CHUNK = E // NW
NROWS = N // NS

_mesh = plsc.VectorSubcoreMesh(core_axis_name="c", subcore_axis_name="s")
_sc_params = pltpu.CompilerParams(needs_layout_passes=False)


NGRP = CHUNK // 16 + 1
CPAD = NGRP * 16


def _gather_half(idx_v, htab, rows_v):
    lanes = jnp.arange(16, dtype=jnp.int32)

    @plsc.parallel_loop(0, NGRP, unroll=4)
    def body(k):
        sv = idx_v[pl.ds(k * 16, 16)]
        base = sv * 8
        eids = (k * 16 + lanes) * 8
        for w in range(8):
            vals = plsc.load_gather(htab, [base + w])
            plsc.store_scatter(rows_v, [eids + w], vals)


@functools.partial(
    pl.kernel,
    mesh=_mesh,
    out_type=[jax.ShapeDtypeStruct((E * 8,), jnp.float32),
              jax.ShapeDtypeStruct((E * 8,), jnp.float32)],
    compiler_params=_sc_params,
    scratch_types=[
        pltpu.VMEM((CPAD + 16,), jnp.int32),
        pltpu.VMEM((N * 8,), jnp.float32),
        pltpu.VMEM((CPAD * 8,), jnp.float32),
    ],
)
def _sc_gather(ha_hbm, hb_hbm, src_hbm, outa_hbm, outb_hbm, idx_v, htab, rows_v):
    wid = lax.axis_index("s") * NC + lax.axis_index("c")
    base = wid * CHUNK
    pltpu.sync_copy(src_hbm.at[pl.ds(base, CHUNK)], idx_v.at[pl.ds(0, CHUNK)])
    idx_v[pl.ds(CHUNK, 16)] = jnp.zeros((16,), jnp.int32)
    pltpu.sync_copy(ha_hbm, htab)
    _gather_half(idx_v, htab, rows_v)
    pltpu.sync_copy(rows_v.at[pl.ds(0, CHUNK * 8)],
                    outa_hbm.at[pl.ds(base * 8, CHUNK * 8)])
    pltpu.sync_copy(hb_hbm, htab)
    _gather_half(idx_v, htab, rows_v)
    pltpu.sync_copy(rows_v.at[pl.ds(0, CHUNK * 8)],
                    outb_hbm.at[pl.ds(base * 8, CHUNK * 8)])


APAD = 640
NPAIR = CHUNK // 2


@functools.partial(
    pl.kernel,
    mesh=_mesh,
    out_type=[jax.ShapeDtypeStruct((NC, APAD, 128), jnp.float32),
              jax.ShapeDtypeStruct((NC, APAD, 128), jnp.float32)],
    compiler_params=_sc_params,
    scratch_types=[
        pltpu.VMEM((CHUNK,), jnp.int32),
        pltpu.VMEM((CHUNK * 4,), jnp.float32),
        pltpu.VMEM((APAD, 128), jnp.float32),
        pltpu.VMEM((APAD,), jnp.int32),
        pltpu.VMEM_SHARED((APAD, 128), jnp.float32),
    ],
)
def _sc_scatter(msga_hbm, msgb_hbm, dst_hbm, zero_hbm, outa_hbm, outb_hbm,
                idx_v, vals_v, acc_v, iota_v, accum_sh):
    c = lax.axis_index("c")
    s = lax.axis_index("s")
    wid = s * NC + c
    base = wid * CHUNK
    lanes = jnp.arange(16, dtype=jnp.int32)
    lo = lanes < 8
    hi = lanes >= 8

    pltpu.sync_copy(dst_hbm.at[pl.ds(base, CHUNK)], idx_v)

    def iota_fill(r, carry):
        iota_v[pl.ds(r * 16, 16)] = r * 16 + lanes
        return carry

    lax.fori_loop(0, APAD // 16, iota_fill, 0)

    for msg_hbm, out_hbm in ((msga_hbm, outa_hbm), (msgb_hbm, outb_hbm)):
        pltpu.sync_copy(zero_hbm, acc_v)
        pltpu.sync_copy(zero_hbm.at[pl.ds(s * 40, 40)],
                        accum_sh.at[pl.ds(s * 40, 40)])
        for sub in range(2):
            pltpu.sync_copy(
                msg_hbm.at[pl.ds(base * 8 + sub * (CHUNK * 4), CHUNK * 4)],
                vals_v)
            e0 = sub * (CHUNK // 2)

            @plsc.parallel_loop(0, NPAIR // 2, unroll=4)
            def pair(k):
                dpair = plsc.load_gather(idx_v, [e0 + 2 * k + (lanes >> 3)])
                a = dpair * 8 + (lanes & 7)
                row = a >> 7
                col = a & 127
                vals = vals_v[pl.ds(k * 16, 16)]
                plsc.addupdate_scatter(acc_v, [row, col], vals, mask=lo)
                plsc.addupdate_scatter(acc_v, [row, col], vals, mask=hi)
        plsc.subcore_barrier()
        pltpu.sync_copy(acc_v, accum_sh.at[iota_v], add=True)
        plsc.subcore_barrier()
        pltpu.sync_copy(accum_sh.at[pl.ds(s * 40, 40)],
                        out_hbm.at[c, pl.ds(s * 40, 40)])
        plsc.subcore_barrier()


def _mlp_body(x_ref, w_ref, b_ref, o_ref):
    y = jnp.dot(x_ref[...], w_ref[...], preferred_element_type=jnp.float32)
    o_ref[...] = jnp.maximum(y + b_ref[...], 0.0)


def _tc_input_mlp(x, W_in, b_in):
    BLK = 2000
    return pl.pallas_call(
        _mlp_body,
        grid=(N // BLK,),
        in_specs=[
            pl.BlockSpec((BLK, 128), lambda i: (i, 0)),
            pl.BlockSpec((128, H), lambda i: (0, 0)),
            pl.BlockSpec((1, H), lambda i: (0, 0)),
        ],
        out_specs=pl.BlockSpec((BLK, H), lambda i: (i, 0)),
        out_shape=jax.ShapeDtypeStruct((N, H), jnp.float32),
    )(x, W_in, b_in.reshape(1, H))


def _msg_body(ea_ref, ga_ref, gb_ref, w1r_ref, b1r_ref, ms_ref, f_ref, b2_ref,
              o_ref):
    ea = ea_ref[...]
    g = jnp.concatenate([ga_ref[...], gb_ref[...]], axis=1)
    tb = jnp.dot(ea, w1r_ref[...], preferred_element_type=jnp.float32)
    tb = jnp.maximum(tb + b1r_ref[...], 0.0)
    u = jnp.dot(g, ms_ref[...], preferred_element_type=jnp.float32)
    msg = jnp.dot(tb * u, f_ref[...], preferred_element_type=jnp.float32)
    msg = msg + jnp.dot(g, b2_ref[...], preferred_element_type=jnp.float32)
    o_ref[...] = msg


def _tc_msg(edge_attr, ga, gb, W1R, b1R, Mstack, F, B2r):
    BLK = 4000
    return pl.pallas_call(
        _msg_body,
        grid=(E // BLK,),
        in_specs=[
            pl.BlockSpec((BLK, 4), lambda i: (i, 0)),
            pl.BlockSpec((BLK, 8), lambda i: (i, 0)),
            pl.BlockSpec((BLK, 8), lambda i: (i, 0)),
            pl.BlockSpec((4, 128), lambda i: (0, 0)),
            pl.BlockSpec((1, 128), lambda i: (0, 0)),
            pl.BlockSpec((H, 128), lambda i: (0, 0)),
            pl.BlockSpec((128, H), lambda i: (0, 0)),
            pl.BlockSpec((H, H), lambda i: (0, 0)),
        ],
        out_specs=pl.BlockSpec((BLK, H), lambda i: (i, 0)),
        out_shape=jax.ShapeDtypeStruct((E, H), jnp.float32),
    )(edge_attr, ga, gb, W1R, b1R, Mstack, F, B2r)


def _node_body(h_ref, pa0_ref, pa1_ref, pb0_ref, pb1_ref, da0_ref, da1_ref,
               wr_ref, br_ref, gm_ref, bt_ref, o_ref):
    h = h_ref[...]
    dh = jnp.maximum(da0_ref[...] + da1_ref[...], 1.0)
    deg = jnp.concatenate([dh, dh], axis=1)
    agg = jnp.concatenate([pa0_ref[...] + pa1_ref[...],
                           pb0_ref[...] + pb1_ref[...]], axis=1) / deg
    u = jnp.dot(h, wr_ref[...], preferred_element_type=jnp.float32)
    u = u + br_ref[...] + agg
    u = u * gm_ref[...] + bt_ref[...]
    o_ref[...] = jnp.maximum(u, 0.0) + h


def _tc_node(h, pa0, pa1, pb0, pb1, da0, da1, Wr, br, gm, bt):
    BLK = 2000
    full = lambda r, c: pl.BlockSpec((r, c), lambda i: (0, 0))
    rowh = lambda: pl.BlockSpec((BLK, 8), lambda i: (i, 0))
    row = lambda: pl.BlockSpec((BLK, H), lambda i: (i, 0))
    return pl.pallas_call(
        _node_body,
        grid=(N // BLK,),
        in_specs=[row(), rowh(), rowh(), rowh(), rowh(), rowh(), rowh(),
                  full(H, H), full(1, H), full(1, H), full(1, H)],
        out_specs=row(),
        out_shape=jax.ShapeDtypeStruct((N, H), jnp.float32),
    )(h, pa0, pa1, pb0, pb1, da0, da1, Wr, br, gm, bt)


def kernel(x, edge_index, edge_attr, W_in, b_in,
           W1_0, b1_0, W2_0, b2_0, Wr_0, br_0, gamma_0, beta_0,
           W1_1, b1_1, W2_1, b2_1, Wr_1, br_1, gamma_1, beta_1,
           W1_2, b1_2, W2_2, b2_2, Wr_2, br_2, gamma_2, beta_2):
    f32 = jnp.float32
    src = edge_index[0]
    dst = edge_index[1]
    zeros_p = jnp.zeros((APAD, 128), f32)
    ones_e = jnp.ones((E * 8,), f32)
    rs = 1.0 / jnp.sqrt(jnp.asarray(1.0 + EPS, f32))
    F = jnp.tile(jnp.eye(H, dtype=f32), (8, 1))

    def unpack(p):
        return p.reshape(APAD * 128)[:N * 8].reshape(N, 8)

    h = _tc_input_mlp(x, W_in, b_in)

    dega, _ = _sc_scatter(ones_e, ones_e, dst, zeros_p)
    da0, da1 = unpack(dega[0]), unpack(dega[1])

    layers = [
        (W1_0, b1_0, W2_0, b2_0, Wr_0, br_0, gamma_0, beta_0),
        (W1_1, b1_1, W2_1, b2_1, Wr_1, br_1, gamma_1, beta_1),
        (W1_2, b1_2, W2_2, b2_2, Wr_2, br_2, gamma_2, beta_2),
    ]
    for (W1, b1, W2, b2, Wr, br, gm, bt) in layers:
        W1R = jnp.repeat(W1, H, axis=1)
        b1R = jnp.repeat(b1, H).reshape(1, 8 * H)
        Mstack = W2.reshape(8, H, H).transpose(1, 0, 2).reshape(H, 8 * H)
        B2r = b2.reshape(H, H)
        ha = h[:, :8].reshape(N * 8)
        hb = h[:, 8:].reshape(N * 8)
        ga, gb = _sc_gather(ha, hb, src)
        msg = _tc_msg(edge_attr, ga.reshape(E, 8), gb.reshape(E, 8),
                      W1R, b1R, Mstack, F, B2r)
        msga = msg[:, :8].reshape(E * 8)
        msgb = msg[:, 8:].reshape(E * 8)
        pa, pb = _sc_scatter(msga, msgb, dst, zeros_p)
        h = _tc_node(h, unpack(pa[0]), unpack(pa[1]),
                     unpack(pb[0]), unpack(pb[1]), da0, da1,
                     Wr, br.reshape(1, H), (gm * rs).reshape(1, H),
                     bt.reshape(1, H))
    return h


# speedup vs baseline: 5.3497x; 2.7608x over previous
import functools

import jax
import jax.numpy as jnp
from jax import lax
from jax.experimental import pallas as pl
from jax.experimental.pallas import tpu as pltpu
from jax.experimental.pallas import tpu_sc as plsc

N = 10000
E = 160000
H = 16
EPS = 1e-5

NC = 2
NS = 16
NW = NC * NS
CHUNK = E // NW
NR = N // 16
ER = E // 8

_mesh = plsc.VectorSubcoreMesh(core_axis_name="c", subcore_axis_name="s")
_sc_params = pltpu.CompilerParams(needs_layout_passes=False)


CPAD = 5008
SUBS = ((0, 156, 2496), (156, 157, 2504))
RWORDS = 157 * 16 * 16


@functools.partial(
    pl.kernel,
    mesh=_mesh,
    out_type=jax.ShapeDtypeStruct((E * 16,), jnp.float32),
    compiler_params=_sc_params,
    scratch_types=[
        pltpu.VMEM((CPAD + 16,), jnp.int32),
        pltpu.VMEM((N * 8,), jnp.float32),
        pltpu.VMEM((RWORDS,), jnp.float32),
    ],
)
def _sc_gather(ha_hbm, hb_hbm, src_hbm, out_hbm, idx_v, htab, rows_v):
    wid = lax.axis_index("s") * NC + lax.axis_index("c")
    base = wid * CHUNK
    lanes = jnp.arange(16, dtype=jnp.int32)
    pltpu.sync_copy(src_hbm.at[pl.ds(base, CHUNK)], idx_v.at[pl.ds(0, CHUNK)])
    idx_v[pl.ds(CHUNK, 16)] = jnp.zeros((16,), jnp.int32)

    for g0, ng, ne in SUBS:
        for half, tab in ((0, ha_hbm), (1, hb_hbm)):
            pltpu.sync_copy(tab, htab)

            @plsc.parallel_loop(0, ng, unroll=4)
            def body(k, g0=g0, half=half):
                sv = idx_v[pl.ds((g0 + k) * 16, 16)]
                addr = sv * 8
                eids = (k * 16 + lanes) * 16 + half * 8
                for w in range(8):
                    vals = plsc.load_gather(htab, [addr + w])
                    plsc.store_scatter(rows_v, [eids + w], vals)

        pltpu.sync_copy(rows_v.at[pl.ds(0, ne * 16)],
                        out_hbm.at[pl.ds((base + g0 * 16) * 16, ne * 16)])


APAD = 640


@functools.partial(
    pl.kernel,
    mesh=_mesh,
    out_type=[jax.ShapeDtypeStruct((NC, APAD, 128), jnp.float32),
              jax.ShapeDtypeStruct((NC, APAD, 128), jnp.float32)],
    compiler_params=_sc_params,
    scratch_types=[
        pltpu.VMEM((CHUNK,), jnp.int32),
        pltpu.VMEM((CHUNK * 4,), jnp.float32),
        pltpu.VMEM((APAD, 128), jnp.float32),
        pltpu.VMEM((APAD,), jnp.int32),
        pltpu.VMEM_SHARED((APAD, 128), jnp.float32),
    ],
)
def _sc_scatter(msg_hbm, dst_hbm, zero_hbm, outa_hbm, outb_hbm,
                idx_v, vals_v, acc_v, iota_v, accum_sh):
    c = lax.axis_index("c")
    s = lax.axis_index("s")
    wid = s * NC + c
    base = wid * CHUNK
    lanes = jnp.arange(16, dtype=jnp.int32)
    lo = lanes < 8
    hi = lanes >= 8

    pltpu.sync_copy(dst_hbm.at[pl.ds(base, CHUNK)], idx_v)

    def iota_fill(r, carry):
        iota_v[pl.ds(r * 16, 16)] = r * 16 + lanes
        return carry

    lax.fori_loop(0, APAD // 16, iota_fill, 0)

    for half, out_hbm in ((0, outa_hbm), (1, outb_hbm)):
        pltpu.sync_copy(zero_hbm, acc_v)
        pltpu.sync_copy(zero_hbm.at[pl.ds(s * 40, 40)],
                        accum_sh.at[pl.ds(s * 40, 40)])
        for sub in range(4):
            pltpu.sync_copy(
                msg_hbm.at[pl.ds((base + sub * 1250) * 16, 1250 * 16)],
                vals_v)
            e0 = sub * 1250

            @plsc.parallel_loop(0, 625, unroll=4)
            def pair(k, e0=e0, half=half):
                dpair = plsc.load_gather(idx_v, [e0 + 2 * k + (lanes >> 3)])
                a = dpair * 8 + (lanes & 7)
                row = a >> 7
                col = a & 127
                vals = plsc.load_gather(
                    vals_v,
                    [k * 32 + (lanes >> 3) * 16 + half * 8 + (lanes & 7)])
                plsc.addupdate_scatter(acc_v, [row, col], vals, mask=lo)
                plsc.addupdate_scatter(acc_v, [row, col], vals, mask=hi)

        plsc.subcore_barrier()
        pltpu.sync_copy(acc_v, accum_sh.at[iota_v], add=True)
        plsc.subcore_barrier()
        pltpu.sync_copy(accum_sh.at[pl.ds(s * 40, 40)],
                        out_hbm.at[c, pl.ds(s * 40, 40)])
        plsc.subcore_barrier()


def _mlp_body(x_ref, w_ref, b_ref, qa_ref, qb_ref, oh_ref, oa_ref, ob_ref):
    y = jnp.dot(x_ref[...], w_ref[...], preferred_element_type=jnp.float32)
    hn = jnp.maximum(y + b_ref[...], 0.0)
    oh_ref[...] = hn
    oa_ref[...] = jnp.dot(hn, qa_ref[...], preferred_element_type=jnp.float32)
    ob_ref[...] = jnp.dot(hn, qb_ref[...], preferred_element_type=jnp.float32)


def _tc_input_mlp(x16, W16, b16, QA, QB):
    return pl.pallas_call(
        _mlp_body,
        out_shape=[jax.ShapeDtypeStruct((NR, 256), jnp.float32),
                   jax.ShapeDtypeStruct((NR, 128), jnp.float32),
                   jax.ShapeDtypeStruct((NR, 128), jnp.float32)],
    )(x16, W16, b16, QA, QB)


def _msg_body(ea_ref, g_ref, a_ref, b1_ref, m_ref, b2_ref, o_ref):
    ea = ea_ref[...]
    g = g_ref[...]
    tb = jnp.dot(ea, a_ref[...], preferred_element_type=jnp.float32)
    tb = jnp.maximum(tb + b1_ref[...], 0.0)
    u = jnp.dot(g, m_ref[...], preferred_element_type=jnp.float32)
    prod = tb * u
    acc = jnp.dot(g, b2_ref[...], preferred_element_type=jnp.float32)
    for b in range(8):
        acc = acc + prod[:, b * 128:(b + 1) * 128]
    o_ref[...] = acc


def _tc_msg(EA8, G8, Astack, b1stack, Mbig, B2big):
    BLK = 400
    return pl.pallas_call(
        _msg_body,
        grid=(ER // BLK,),
        in_specs=[
            pl.BlockSpec((BLK, 32), lambda i: (i, 0)),
            pl.BlockSpec((BLK, 128), lambda i: (i, 0)),
            pl.BlockSpec((32, 1024), lambda i: (0, 0)),
            pl.BlockSpec((1, 1024), lambda i: (0, 0)),
            pl.BlockSpec((128, 1024), lambda i: (0, 0)),
            pl.BlockSpec((128, 128), lambda i: (0, 0)),
        ],
        out_specs=pl.BlockSpec((BLK, 128), lambda i: (i, 0)),
        out_shape=jax.ShapeDtypeStruct((ER, 128), jnp.float32),
    )(EA8, G8, Astack, b1stack, Mbig, B2big)


def _node_body(h_ref, pa0, pa1, pb0, pb1, da0, da1, wr_ref, br_ref, gm_ref,
               bt_ref, pam_ref, pbm_ref, qa_ref, qb_ref,
               oh_ref, oa_ref, ob_ref):
    f32 = jnp.float32
    h = h_ref[...]
    pam = pam_ref[...]
    pbm = pbm_ref[...]
    agg_a = pa0[...] + pa1[...]
    agg_b = pb0[...] + pb1[...]
    agg = (jnp.dot(agg_a, pam, preferred_element_type=f32)
           + jnp.dot(agg_b, pbm, preferred_element_type=f32))
    d_a = da0[...] + da1[...]
    deg = jnp.dot(d_a, pam + pbm, preferred_element_type=f32)
    deg = jnp.maximum(deg, 1.0)
    u = jnp.dot(h, wr_ref[...], preferred_element_type=f32)
    u = u + br_ref[...] + agg / deg
    u = u * gm_ref[...] + bt_ref[...]
    hn = jnp.maximum(u, 0.0) + h
    oh_ref[...] = hn
    oa_ref[...] = jnp.dot(hn, qa_ref[...], preferred_element_type=f32)
    ob_ref[...] = jnp.dot(hn, qb_ref[...], preferred_element_type=f32)


def _tc_node(h16, pa0, pa1, pb0, pb1, da0, da1, Wr16, br16, gm16, bt16,
             PA, PB, QA, QB):
    return pl.pallas_call(
        _node_body,
        out_shape=[jax.ShapeDtypeStruct((NR, 256), jnp.float32),
                   jax.ShapeDtypeStruct((NR, 128), jnp.float32),
                   jax.ShapeDtypeStruct((NR, 128), jnp.float32)],
    )(h16, pa0, pa1, pb0, pb1, da0, da1, Wr16, br16, gm16, bt16,
      PA, PB, QA, QB)


def kernel(x, edge_index, edge_attr, W_in, b_in,
           W1_0, b1_0, W2_0, b2_0, Wr_0, br_0, gamma_0, beta_0,
           W1_1, b1_1, W2_1, b2_1, Wr_1, br_1, gamma_1, beta_1,
           W1_2, b1_2, W2_2, b2_2, Wr_2, br_2, gamma_2, beta_2):
    f32 = jnp.float32
    src = edge_index[0]
    dst = edge_index[1]
    zeros_p = jnp.zeros((APAD, 128), f32)
    ones_e = jnp.ones((E * 16,), f32)
    rs = 1.0 / jnp.sqrt(jnp.asarray(1.0 + EPS, f32))

    eye8 = jnp.eye(8, dtype=f32)
    eye16 = jnp.eye(16, dtype=f32)
    PA = jnp.einsum('ji,cd->cjdi', jnp.eye(8, 16, dtype=f32),
                    eye16).reshape(128, 256)
    PB = jnp.einsum('ji,cd->cjdi', jnp.eye(8, 16, k=8, dtype=f32),
                    eye16).reshape(128, 256)
    QA = jnp.einsum('ij,cd->cidj', jnp.eye(16, 8, dtype=f32),
                    eye16).reshape(256, 128)
    QB = jnp.einsum('ij,cd->cidj', jnp.eye(16, 8, k=-8, dtype=f32),
                    eye16).reshape(256, 128)

    W16 = jnp.einsum('do,ce->cdeo', W_in, eye16).reshape(16 * 128, 256)
    b16 = jnp.tile(b_in, 16).reshape(1, 256)

    x16 = x.reshape(NR, 16 * 128)
    EA8 = edge_attr.reshape(ER, 32)

    h16, hap, hbp = _tc_input_mlp(x16, W16, b16, QA, QB)

    dega, _ = _sc_scatter(ones_e, dst, zeros_p)
    da0 = dega[0, :NR]
    da1 = dega[1, :NR]

    layers = [
        (W1_0, b1_0, W2_0, b2_0, Wr_0, br_0, gamma_0, beta_0),
        (W1_1, b1_1, W2_1, b2_1, Wr_1, br_1, gamma_1, beta_1),
        (W1_2, b1_2, W2_2, b2_2, Wr_2, br_2, gamma_2, beta_2),
    ]
    for (W1, b1, W2, b2, Wr, br, gm, bt) in layers:
        Astack = (W1[None, :, :, None, None] * eye8[:, None, None, :, None])
        Astack = jnp.broadcast_to(Astack, (8, 4, 8, 8, 16)).reshape(32, 1024)
        b1stack = jnp.repeat(b1, 128).reshape(1, 1024)
        W2r = W2.reshape(8, H, H)
        Mbig = jnp.einsum('bio,cd->cibdo', W2r, eye8).reshape(128, 1024)
        B2big = jnp.einsum('io,cd->cido', b2.reshape(H, H),
                           eye8).reshape(128, 128)
        Wr16 = jnp.einsum('io,cd->cido', Wr, eye16).reshape(256, 256)
        br16 = jnp.tile(br, 16).reshape(1, 256)
        gm16 = jnp.tile(gm * rs, 16).reshape(1, 256)
        bt16 = jnp.tile(bt, 16).reshape(1, 256)

        g8 = _sc_gather(hap.reshape(N * 8), hbp.reshape(N * 8), src)
        msg8 = _tc_msg(EA8, g8.reshape(ER, 128), Astack, b1stack, Mbig, B2big)
        pa, pb = _sc_scatter(msg8.reshape(E * 16), dst, zeros_p)
        h16, hap, hbp = _tc_node(h16, pa[0, :NR], pa[1, :NR],
                                 pb[0, :NR], pb[1, :NR], da0, da1,
                                 Wr16, br16, gm16, bt16, PA, PB, QA, QB)
    return h16.reshape(N, H)


# speedup vs baseline: 5.5074x; 1.0295x over previous
import functools

import jax
import jax.numpy as jnp
from jax import lax
from jax.experimental import pallas as pl
from jax.experimental.pallas import tpu as pltpu
from jax.experimental.pallas import tpu_sc as plsc

N = 10000
E = 160000
H = 16
EPS = 1e-5

NC = 2
NS = 16
NW = NC * NS
CHUNK = E // NW
NR = N // 16
ER = E // 8

_mesh = plsc.VectorSubcoreMesh(core_axis_name="c", subcore_axis_name="s")
_sc_params = pltpu.CompilerParams(needs_layout_passes=False)


CPAD = 5008
SUBS = ((0, 156, 2496), (156, 157, 2504))
RWORDS = 157 * 16 * 16


@functools.partial(
    pl.kernel,
    mesh=_mesh,
    out_type=jax.ShapeDtypeStruct((E * 16,), jnp.float32),
    compiler_params=_sc_params,
    scratch_types=[
        pltpu.VMEM((CPAD + 16,), jnp.int32),
        pltpu.VMEM((N * 8,), jnp.float32),
        pltpu.VMEM((RWORDS,), jnp.float32),
    ],
)
def _sc_gather(ha_hbm, hb_hbm, src_hbm, out_hbm, idx_v, htab, rows_v):
    wid = lax.axis_index("s") * NC + lax.axis_index("c")
    base = wid * CHUNK
    lanes = jnp.arange(16, dtype=jnp.int32)
    pltpu.sync_copy(src_hbm.at[pl.ds(base, CHUNK)], idx_v.at[pl.ds(0, CHUNK)])
    idx_v[pl.ds(CHUNK, 16)] = jnp.zeros((16,), jnp.int32)

    for g0, ng, ne in SUBS:
        for half, tab in ((0, ha_hbm), (1, hb_hbm)):
            pltpu.sync_copy(tab, htab)

            @plsc.parallel_loop(0, ng, unroll=4)
            def body(k, g0=g0, half=half):
                sv = idx_v[pl.ds((g0 + k) * 16, 16)]
                addr = sv * 8
                eids = (k * 16 + lanes) * 16 + half * 8
                for w in range(8):
                    vals = plsc.load_gather(htab, [addr + w])
                    plsc.store_scatter(rows_v, [eids + w], vals)

        pltpu.sync_copy(rows_v.at[pl.ds(0, ne * 16)],
                        out_hbm.at[pl.ds((base + g0 * 16) * 16, ne * 16)])


APAD = 640
NPAIR = CHUNK // 2


@functools.partial(
    pl.kernel,
    mesh=_mesh,
    out_type=[jax.ShapeDtypeStruct((NC, APAD, 128), jnp.float32),
              jax.ShapeDtypeStruct((NC, APAD, 128), jnp.float32),
              jax.ShapeDtypeStruct((NC, APAD, 128), jnp.float32)],
    compiler_params=_sc_params,
    scratch_types=[
        pltpu.VMEM((CHUNK,), jnp.int32),
        pltpu.VMEM((CHUNK * 4,), jnp.float32),
        pltpu.VMEM((APAD, 128), jnp.float32),
        pltpu.VMEM((APAD,), jnp.int32),
        pltpu.VMEM_SHARED((APAD, 128), jnp.float32),
    ],
)
def _sc_scatter(msg_hbm, dst_hbm, zero_hbm, outa_hbm, outb_hbm, outd_hbm,
                idx_v, vals_v, acc_v, iota_v, accum_sh):
    c = lax.axis_index("c")
    s = lax.axis_index("s")
    wid = s * NC + c
    base = wid * CHUNK
    lanes = jnp.arange(16, dtype=jnp.int32)
    lo = lanes < 8
    hi = lanes >= 8
    zero16 = jnp.zeros((16,), jnp.float32)

    pltpu.sync_copy(dst_hbm.at[pl.ds(base, CHUNK)], idx_v)

    def iota_fill(r, carry):
        iota_v[pl.ds(r * 16, 16)] = r * 16 + lanes
        return carry

    lax.fori_loop(0, APAD // 16, iota_fill, 0)

    def one_pass(out_hbm, body_of_pass):
        @plsc.parallel_loop(0, APAD, unroll=8)
        def zfill(r):
            for cc in range(8):
                acc_v[r, pl.ds(cc * 16, 16)] = zero16

        pltpu.sync_copy(zero_hbm.at[pl.ds(s * 40, 40)],
                        accum_sh.at[pl.ds(s * 40, 40)])
        body_of_pass()
        plsc.subcore_barrier()
        pltpu.sync_copy(acc_v, accum_sh.at[iota_v], add=True)
        plsc.subcore_barrier()
        pltpu.sync_copy(accum_sh.at[pl.ds(s * 40, 40)],
                        out_hbm.at[c, pl.ds(s * 40, 40)])
        plsc.subcore_barrier()

    for half, out_hbm in ((0, outa_hbm), (1, outb_hbm)):
        def value_pass(half=half):
            for sub in range(4):
                pltpu.sync_copy(
                    msg_hbm.at[pl.ds((base + sub * 1250) * 16, 1250 * 16)],
                    vals_v)
                e0 = sub * 1250

                @plsc.parallel_loop(0, 625, unroll=4)
                def pair(k, e0=e0, half=half):
                    dpair = plsc.load_gather(idx_v,
                                             [e0 + 2 * k + (lanes >> 3)])
                    a = dpair * 8 + (lanes & 7)
                    vals = plsc.load_gather(
                        vals_v,
                        [k * 32 + (lanes >> 3) * 16 + half * 8 + (lanes & 7)])
                    plsc.addupdate_scatter(acc_v, [a >> 7, a & 127], vals,
                                           mask=lo)
                    plsc.addupdate_scatter(acc_v, [a >> 7, a & 127], vals,
                                           mask=hi)

        one_pass(out_hbm, value_pass)

    def ones_pass():
        ones16 = jnp.ones((16,), jnp.float32)

        @plsc.parallel_loop(0, NPAIR, unroll=4)
        def pair(k):
            dpair = plsc.load_gather(idx_v, [2 * k + (lanes >> 3)])
            a = dpair * 8 + (lanes & 7)
            plsc.addupdate_scatter(acc_v, [a >> 7, a & 127], ones16, mask=lo)
            plsc.addupdate_scatter(acc_v, [a >> 7, a & 127], ones16, mask=hi)

    one_pass(outd_hbm, ones_pass)


def _mlp_body(x_ref, w_ref, b_ref, qa_ref, qb_ref, oh_ref, oa_ref, ob_ref):
    y = jnp.dot(x_ref[...], w_ref[...], preferred_element_type=jnp.float32)
    hn = jnp.maximum(y + b_ref[...], 0.0)
    oh_ref[...] = hn
    oa_ref[...] = jnp.dot(hn, qa_ref[...], preferred_element_type=jnp.float32)
    ob_ref[...] = jnp.dot(hn, qb_ref[...], preferred_element_type=jnp.float32)


def _tc_input_mlp(x16, W16, b16, QA, QB):
    return pl.pallas_call(
        _mlp_body,
        out_shape=[jax.ShapeDtypeStruct((NR, 256), jnp.float32),
                   jax.ShapeDtypeStruct((NR, 128), jnp.float32),
                   jax.ShapeDtypeStruct((NR, 128), jnp.float32)],
    )(x16, W16, b16, QA, QB)


def _msg_body(ea_ref, g_ref, a_ref, b1_ref, m_ref, b2_ref, o_ref):
    ea = ea_ref[...]
    g = g_ref[...]
    tb = jnp.dot(ea, a_ref[...], preferred_element_type=jnp.float32)
    tb = jnp.maximum(tb + b1_ref[...], 0.0)
    u = jnp.dot(g, m_ref[...], preferred_element_type=jnp.float32)
    prod = tb * u
    acc = jnp.dot(g, b2_ref[...], preferred_element_type=jnp.float32)
    for b in range(8):
        acc = acc + prod[:, b * 128:(b + 1) * 128]
    o_ref[...] = acc


def _tc_msg(EA8, G8, Astack, b1stack, Mbig, B2big):
    BLK = 400
    return pl.pallas_call(
        _msg_body,
        grid=(ER // BLK,),
        in_specs=[
            pl.BlockSpec((BLK, 32), lambda i: (i, 0)),
            pl.BlockSpec((BLK, 128), lambda i: (i, 0)),
            pl.BlockSpec((32, 1024), lambda i: (0, 0)),
            pl.BlockSpec((1, 1024), lambda i: (0, 0)),
            pl.BlockSpec((128, 1024), lambda i: (0, 0)),
            pl.BlockSpec((128, 128), lambda i: (0, 0)),
        ],
        out_specs=pl.BlockSpec((BLK, 128), lambda i: (i, 0)),
        out_shape=jax.ShapeDtypeStruct((ER, 128), jnp.float32),
    )(EA8, G8, Astack, b1stack, Mbig, B2big)


def _node_body(h_ref, pa0, pa1, pb0, pb1, da0, da1, wr_ref, br_ref, gm_ref,
               bt_ref, pam_ref, pbm_ref, qa_ref, qb_ref,
               oh_ref, oa_ref, ob_ref):
    f32 = jnp.float32
    h = h_ref[...]
    pam = pam_ref[...]
    pbm = pbm_ref[...]
    agg_a = pa0[...] + pa1[...]
    agg_b = pb0[...] + pb1[...]
    agg = (jnp.dot(agg_a, pam, preferred_element_type=f32)
           + jnp.dot(agg_b, pbm, preferred_element_type=f32))
    d_a = da0[...] + da1[...]
    deg = jnp.dot(d_a, pam + pbm, preferred_element_type=f32)
    deg = jnp.maximum(deg, 1.0)
    u = jnp.dot(h, wr_ref[...], preferred_element_type=f32)
    u = u + br_ref[...] + agg / deg
    u = u * gm_ref[...] + bt_ref[...]
    hn = jnp.maximum(u, 0.0) + h
    oh_ref[...] = hn
    oa_ref[...] = jnp.dot(hn, qa_ref[...], preferred_element_type=f32)
    ob_ref[...] = jnp.dot(hn, qb_ref[...], preferred_element_type=f32)


def _tc_node(h16, pa0, pa1, pb0, pb1, da0, da1, Wr16, br16, gm16, bt16,
             PA, PB, QA, QB):
    return pl.pallas_call(
        _node_body,
        out_shape=[jax.ShapeDtypeStruct((NR, 256), jnp.float32),
                   jax.ShapeDtypeStruct((NR, 128), jnp.float32),
                   jax.ShapeDtypeStruct((NR, 128), jnp.float32)],
    )(h16, pa0, pa1, pb0, pb1, da0, da1, Wr16, br16, gm16, bt16,
      PA, PB, QA, QB)


def kernel(x, edge_index, edge_attr, W_in, b_in,
           W1_0, b1_0, W2_0, b2_0, Wr_0, br_0, gamma_0, beta_0,
           W1_1, b1_1, W2_1, b2_1, Wr_1, br_1, gamma_1, beta_1,
           W1_2, b1_2, W2_2, b2_2, Wr_2, br_2, gamma_2, beta_2):
    f32 = jnp.float32
    src = edge_index[0]
    dst = edge_index[1]
    zeros_p = jnp.zeros((APAD, 128), f32)
    rs = 1.0 / jnp.sqrt(jnp.asarray(1.0 + EPS, f32))

    eye8 = jnp.eye(8, dtype=f32)
    eye16 = jnp.eye(16, dtype=f32)
    PA = jnp.einsum('ji,cd->cjdi', jnp.eye(8, 16, dtype=f32),
                    eye16).reshape(128, 256)
    PB = jnp.einsum('ji,cd->cjdi', jnp.eye(8, 16, k=8, dtype=f32),
                    eye16).reshape(128, 256)
    QA = jnp.einsum('ij,cd->cidj', jnp.eye(16, 8, dtype=f32),
                    eye16).reshape(256, 128)
    QB = jnp.einsum('ij,cd->cidj', jnp.eye(16, 8, k=-8, dtype=f32),
                    eye16).reshape(256, 128)

    W16 = jnp.einsum('do,ce->cdeo', W_in, eye16).reshape(16 * 128, 256)
    b16 = jnp.tile(b_in, 16).reshape(1, 256)

    x16 = x.reshape(NR, 16 * 128)
    EA8 = edge_attr.reshape(ER, 32)

    h16, hap, hbp = _tc_input_mlp(x16, W16, b16, QA, QB)
    da0 = da1 = None

    layers = [
        (W1_0, b1_0, W2_0, b2_0, Wr_0, br_0, gamma_0, beta_0),
        (W1_1, b1_1, W2_1, b2_1, Wr_1, br_1, gamma_1, beta_1),
        (W1_2, b1_2, W2_2, b2_2, Wr_2, br_2, gamma_2, beta_2),
    ]
    for (W1, b1, W2, b2, Wr, br, gm, bt) in layers:
        Astack = (W1[None, :, :, None, None] * eye8[:, None, None, :, None])
        Astack = jnp.broadcast_to(Astack, (8, 4, 8, 8, 16)).reshape(32, 1024)
        b1stack = jnp.repeat(b1, 128).reshape(1, 1024)
        W2r = W2.reshape(8, H, H)
        Mbig = jnp.einsum('bio,cd->cibdo', W2r, eye8).reshape(128, 1024)
        B2big = jnp.einsum('io,cd->cido', b2.reshape(H, H),
                           eye8).reshape(128, 128)
        Wr16 = jnp.einsum('io,cd->cido', Wr, eye16).reshape(256, 256)
        br16 = jnp.tile(br, 16).reshape(1, 256)
        gm16 = jnp.tile(gm * rs, 16).reshape(1, 256)
        bt16 = jnp.tile(bt, 16).reshape(1, 256)

        g8 = _sc_gather(hap.reshape(N * 8), hbp.reshape(N * 8), src)
        msg8 = _tc_msg(EA8, g8.reshape(ER, 128), Astack, b1stack, Mbig, B2big)
        pa, pb, pd = _sc_scatter(msg8.reshape(E * 16), dst, zeros_p)
        if da0 is None:
            da0, da1 = pd[0, :NR], pd[1, :NR]
        h16, hap, hbp = _tc_node(h16, pa[0, :NR], pa[1, :NR],
                                 pb[0, :NR], pb[1, :NR], da0, da1,
                                 Wr16, br16, gm16, bt16, PA, PB, QA, QB)
    return h16.reshape(N, H)


# speedup vs baseline: 5.6388x; 1.0238x over previous
import functools

import jax
import jax.numpy as jnp
from jax import lax
from jax.experimental import pallas as pl
from jax.experimental.pallas import tpu as pltpu
from jax.experimental.pallas import tpu_sc as plsc

N = 10000
E = 160000
H = 16
EPS = 1e-5

NC = 2
NS = 16
NW = NC * NS
CHUNK = E // NW
NR = N // 16
ER = E // 8

_mesh = plsc.VectorSubcoreMesh(core_axis_name="c", subcore_axis_name="s")
_sc_params = pltpu.CompilerParams(needs_layout_passes=False)


CPAD = 5008
SUBS = ((0, 156, 2496), (156, 157, 2504))
RWORDS = 157 * 16 * 16


@functools.partial(
    pl.kernel,
    mesh=_mesh,
    out_type=jax.ShapeDtypeStruct((E * 16,), jnp.float32),
    compiler_params=_sc_params,
    scratch_types=[
        pltpu.VMEM((CPAD + 16,), jnp.int32),
        pltpu.VMEM((N * 8,), jnp.float32),
        pltpu.VMEM((RWORDS,), jnp.float32),
    ],
)
def _sc_gather(ha_hbm, hb_hbm, src_hbm, out_hbm, idx_v, htab, rows_v):
    wid = lax.axis_index("s") * NC + lax.axis_index("c")
    base = wid * CHUNK
    lanes = jnp.arange(16, dtype=jnp.int32)
    pltpu.sync_copy(src_hbm.at[pl.ds(base, CHUNK)], idx_v.at[pl.ds(0, CHUNK)])
    idx_v[pl.ds(CHUNK, 16)] = jnp.zeros((16,), jnp.int32)

    for g0, ng, ne in SUBS:
        for half, tab in ((0, ha_hbm), (1, hb_hbm)):
            pltpu.sync_copy(tab, htab)

            @plsc.parallel_loop(0, ng, unroll=8)
            def body(k, g0=g0, half=half):
                sv = idx_v[pl.ds((g0 + k) * 16, 16)]
                addr = sv * 8
                eids = (k * 16 + lanes) * 16 + half * 8
                for w in range(8):
                    vals = plsc.load_gather(htab, [addr + w])
                    plsc.store_scatter(rows_v, [eids + w], vals)

        pltpu.sync_copy(rows_v.at[pl.ds(0, ne * 16)],
                        out_hbm.at[pl.ds((base + g0 * 16) * 16, ne * 16)])


APAD = 640
NPAIR = CHUNK // 2


def _make_scatter(with_deg):
  n_out = 3 if with_deg else 2

  @functools.partial(
      pl.kernel,
      mesh=_mesh,
      out_type=[jax.ShapeDtypeStruct((NC, APAD, 128), jnp.float32)] * n_out,
      compiler_params=_sc_params,
      scratch_types=[
          pltpu.VMEM((CHUNK,), jnp.int32),
          pltpu.VMEM((CHUNK * 4,), jnp.float32),
          pltpu.VMEM((APAD, 128), jnp.float32),
          pltpu.VMEM((APAD,), jnp.int32),
          pltpu.VMEM_SHARED((APAD, 128), jnp.float32),
      ],
  )
  def _sc_scatter(msg_hbm, dst_hbm, zero_hbm, *rest):
    if with_deg:
        outa_hbm, outb_hbm, outd_hbm = rest[:3]
    else:
        outa_hbm, outb_hbm = rest[:2]
    idx_v, vals_v, acc_v, iota_v, accum_sh = rest[n_out:]
    c = lax.axis_index("c")
    s = lax.axis_index("s")
    wid = s * NC + c
    base = wid * CHUNK
    lanes = jnp.arange(16, dtype=jnp.int32)
    lo = lanes < 8
    hi = lanes >= 8
    zero16 = jnp.zeros((16,), jnp.float32)

    pltpu.sync_copy(dst_hbm.at[pl.ds(base, CHUNK)], idx_v)

    def iota_fill(r, carry):
        iota_v[pl.ds(r * 16, 16)] = r * 16 + lanes
        return carry

    lax.fori_loop(0, APAD // 16, iota_fill, 0)

    def one_pass(out_hbm, body_of_pass):
        @plsc.parallel_loop(0, APAD, unroll=8)
        def zfill(r):
            for cc in range(8):
                acc_v[r, pl.ds(cc * 16, 16)] = zero16

        pltpu.sync_copy(zero_hbm.at[pl.ds(s * 40, 40)],
                        accum_sh.at[pl.ds(s * 40, 40)])
        body_of_pass()
        plsc.subcore_barrier()
        pltpu.sync_copy(acc_v, accum_sh.at[iota_v], add=True)
        plsc.subcore_barrier()
        pltpu.sync_copy(accum_sh.at[pl.ds(s * 40, 40)],
                        out_hbm.at[c, pl.ds(s * 40, 40)])
        plsc.subcore_barrier()

    for half, out_hbm in ((0, outa_hbm), (1, outb_hbm)):
        def value_pass(half=half):
            for sub in range(4):
                pltpu.sync_copy(
                    msg_hbm.at[pl.ds((base + sub * 1250) * 16, 1250 * 16)],
                    vals_v)
                e0 = sub * 1250

                @plsc.parallel_loop(0, 625, unroll=4)
                def pair(k, e0=e0, half=half):
                    dpair = plsc.load_gather(idx_v,
                                             [e0 + 2 * k + (lanes >> 3)])
                    a = dpair * 8 + (lanes & 7)
                    vals = plsc.load_gather(
                        vals_v,
                        [k * 32 + (lanes >> 3) * 16 + half * 8 + (lanes & 7)])
                    plsc.addupdate_scatter(acc_v, [a >> 7, a & 127], vals,
                                           mask=lo)
                    plsc.addupdate_scatter(acc_v, [a >> 7, a & 127], vals,
                                           mask=hi)

        one_pass(out_hbm, value_pass)

    if with_deg:
        def ones_pass():
            ones16 = jnp.ones((16,), jnp.float32)

            @plsc.parallel_loop(0, NPAIR, unroll=4)
            def pair(k):
                dpair = plsc.load_gather(idx_v, [2 * k + (lanes >> 3)])
                a = dpair * 8 + (lanes & 7)
                plsc.addupdate_scatter(acc_v, [a >> 7, a & 127], ones16,
                                       mask=lo)
                plsc.addupdate_scatter(acc_v, [a >> 7, a & 127], ones16,
                                       mask=hi)

        one_pass(outd_hbm, ones_pass)

  return _sc_scatter


_sc_scatter_deg = _make_scatter(True)
_sc_scatter_nod = _make_scatter(False)


def _mlp_body(x_ref, w_ref, b_ref, qa_ref, qb_ref, oh_ref, oa_ref, ob_ref):
    y = jnp.dot(x_ref[...], w_ref[...], preferred_element_type=jnp.float32)
    hn = jnp.maximum(y + b_ref[...], 0.0)
    oh_ref[...] = hn
    oa_ref[...] = jnp.dot(hn, qa_ref[...], preferred_element_type=jnp.float32)
    ob_ref[...] = jnp.dot(hn, qb_ref[...], preferred_element_type=jnp.float32)


def _tc_input_mlp(x16, W16, b16, QA, QB):
    return pl.pallas_call(
        _mlp_body,
        out_shape=[jax.ShapeDtypeStruct((NR, 256), jnp.float32),
                   jax.ShapeDtypeStruct((NR, 128), jnp.float32),
                   jax.ShapeDtypeStruct((NR, 128), jnp.float32)],
    )(x16, W16, b16, QA, QB)


def _msg_body(ea_ref, g_ref, a_ref, b1_ref, m_ref, b2_ref, o_ref):
    ea = ea_ref[...]
    g = g_ref[...]
    tb = jnp.dot(ea, a_ref[...], preferred_element_type=jnp.float32)
    tb = jnp.maximum(tb + b1_ref[...], 0.0)
    u = jnp.dot(g, m_ref[...], preferred_element_type=jnp.float32)
    prod = tb * u
    acc = jnp.dot(g, b2_ref[...], preferred_element_type=jnp.float32)
    for b in range(8):
        acc = acc + prod[:, b * 128:(b + 1) * 128]
    o_ref[...] = acc


def _tc_msg(EA8, G8, Astack, b1stack, Mbig, B2big):
    BLK = 400
    return pl.pallas_call(
        _msg_body,
        grid=(ER // BLK,),
        in_specs=[
            pl.BlockSpec((BLK, 32), lambda i: (i, 0)),
            pl.BlockSpec((BLK, 128), lambda i: (i, 0)),
            pl.BlockSpec((32, 1024), lambda i: (0, 0)),
            pl.BlockSpec((1, 1024), lambda i: (0, 0)),
            pl.BlockSpec((128, 1024), lambda i: (0, 0)),
            pl.BlockSpec((128, 128), lambda i: (0, 0)),
        ],
        out_specs=pl.BlockSpec((BLK, 128), lambda i: (i, 0)),
        out_shape=jax.ShapeDtypeStruct((ER, 128), jnp.float32),
    )(EA8, G8, Astack, b1stack, Mbig, B2big)


def _node_body(h_ref, pa0, pa1, pb0, pb1, da0, da1, wr_ref, br_ref, gm_ref,
               bt_ref, pam_ref, pbm_ref, qa_ref, qb_ref,
               oh_ref, oa_ref, ob_ref):
    f32 = jnp.float32
    h = h_ref[...]
    pam = pam_ref[...]
    pbm = pbm_ref[...]
    agg_a = pa0[...] + pa1[...]
    agg_b = pb0[...] + pb1[...]
    agg = (jnp.dot(agg_a, pam, preferred_element_type=f32)
           + jnp.dot(agg_b, pbm, preferred_element_type=f32))
    d_a = da0[...] + da1[...]
    deg = jnp.dot(d_a, pam + pbm, preferred_element_type=f32)
    deg = jnp.maximum(deg, 1.0)
    u = jnp.dot(h, wr_ref[...], preferred_element_type=f32)
    u = u + br_ref[...] + agg / deg
    u = u * gm_ref[...] + bt_ref[...]
    hn = jnp.maximum(u, 0.0) + h
    oh_ref[...] = hn
    oa_ref[...] = jnp.dot(hn, qa_ref[...], preferred_element_type=f32)
    ob_ref[...] = jnp.dot(hn, qb_ref[...], preferred_element_type=f32)


def _tc_node(h16, pa0, pa1, pb0, pb1, da0, da1, Wr16, br16, gm16, bt16,
             PA, PB, QA, QB):
    return pl.pallas_call(
        _node_body,
        out_shape=[jax.ShapeDtypeStruct((NR, 256), jnp.float32),
                   jax.ShapeDtypeStruct((NR, 128), jnp.float32),
                   jax.ShapeDtypeStruct((NR, 128), jnp.float32)],
    )(h16, pa0, pa1, pb0, pb1, da0, da1, Wr16, br16, gm16, bt16,
      PA, PB, QA, QB)


def kernel(x, edge_index, edge_attr, W_in, b_in,
           W1_0, b1_0, W2_0, b2_0, Wr_0, br_0, gamma_0, beta_0,
           W1_1, b1_1, W2_1, b2_1, Wr_1, br_1, gamma_1, beta_1,
           W1_2, b1_2, W2_2, b2_2, Wr_2, br_2, gamma_2, beta_2):
    f32 = jnp.float32
    src = edge_index[0]
    dst = edge_index[1]
    zeros_p = jnp.zeros((APAD, 128), f32)
    rs = 1.0 / jnp.sqrt(jnp.asarray(1.0 + EPS, f32))

    eye8 = jnp.eye(8, dtype=f32)
    eye16 = jnp.eye(16, dtype=f32)
    PA = jnp.einsum('ji,cd->cjdi', jnp.eye(8, 16, dtype=f32),
                    eye16).reshape(128, 256)
    PB = jnp.einsum('ji,cd->cjdi', jnp.eye(8, 16, k=8, dtype=f32),
                    eye16).reshape(128, 256)
    QA = jnp.einsum('ij,cd->cidj', jnp.eye(16, 8, dtype=f32),
                    eye16).reshape(256, 128)
    QB = jnp.einsum('ij,cd->cidj', jnp.eye(16, 8, k=-8, dtype=f32),
                    eye16).reshape(256, 128)

    W16 = jnp.einsum('do,ce->cdeo', W_in, eye16).reshape(16 * 128, 256)
    b16 = jnp.tile(b_in, 16).reshape(1, 256)

    x16 = x.reshape(NR, 16 * 128)
    EA8 = edge_attr.reshape(ER, 32)

    h16, hap, hbp = _tc_input_mlp(x16, W16, b16, QA, QB)
    da0 = da1 = None

    layers = [
        (W1_0, b1_0, W2_0, b2_0, Wr_0, br_0, gamma_0, beta_0),
        (W1_1, b1_1, W2_1, b2_1, Wr_1, br_1, gamma_1, beta_1),
        (W1_2, b1_2, W2_2, b2_2, Wr_2, br_2, gamma_2, beta_2),
    ]
    for (W1, b1, W2, b2, Wr, br, gm, bt) in layers:
        Astack = (W1[None, :, :, None, None] * eye8[:, None, None, :, None])
        Astack = jnp.broadcast_to(Astack, (8, 4, 8, 8, 16)).reshape(32, 1024)
        b1stack = jnp.repeat(b1, 128).reshape(1, 1024)
        W2r = W2.reshape(8, H, H)
        Mbig = jnp.einsum('bio,cd->cibdo', W2r, eye8).reshape(128, 1024)
        B2big = jnp.einsum('io,cd->cido', b2.reshape(H, H),
                           eye8).reshape(128, 128)
        Wr16 = jnp.einsum('io,cd->cido', Wr, eye16).reshape(256, 256)
        br16 = jnp.tile(br, 16).reshape(1, 256)
        gm16 = jnp.tile(gm * rs, 16).reshape(1, 256)
        bt16 = jnp.tile(bt, 16).reshape(1, 256)

        g8 = _sc_gather(hap.reshape(N * 8), hbp.reshape(N * 8), src)
        msg8 = _tc_msg(EA8, g8.reshape(ER, 128), Astack, b1stack, Mbig, B2big)
        if da0 is None:
            pa, pb, pd = _sc_scatter_deg(msg8.reshape(E * 16), dst, zeros_p)
            da0, da1 = pd[0, :NR], pd[1, :NR]
        else:
            pa, pb = _sc_scatter_nod(msg8.reshape(E * 16), dst, zeros_p)
        h16, hap, hbp = _tc_node(h16, pa[0, :NR], pa[1, :NR],
                                 pb[0, :NR], pb[1, :NR], da0, da1,
                                 Wr16, br16, gm16, bt16, PA, PB, QA, QB)
    return h16.reshape(N, H)


# speedup vs baseline: 5.7877x; 1.0264x over previous
import functools

import jax
import jax.numpy as jnp
from jax import lax
from jax.experimental import pallas as pl
from jax.experimental.pallas import tpu as pltpu
from jax.experimental.pallas import tpu_sc as plsc

N = 10000
E = 160000
H = 16
EPS = 1e-5

NC = 2
NS = 16
NW = NC * NS
CHUNK = E // NW
NR = N // 16
ER = E // 8

_mesh = plsc.VectorSubcoreMesh(core_axis_name="c", subcore_axis_name="s")
_sc_params = pltpu.CompilerParams(needs_layout_passes=False)


ES = E // 2
CH2 = ES // NW
NG2 = 157
RW2 = NG2 * 16 * 16


def _make_gather(eoff):
  @functools.partial(
      pl.kernel,
      mesh=_mesh,
      out_type=jax.ShapeDtypeStruct((ES * 16,), jnp.float32),
      compiler_params=_sc_params,
      scratch_types=[
          pltpu.VMEM((CH2 + 20,), jnp.int32),
          pltpu.VMEM((N * 8,), jnp.float32),
          pltpu.VMEM((RW2,), jnp.float32),
      ],
  )
  def _sc_gather(ha_hbm, hb_hbm, src_hbm, out_hbm, idx_v, htab, rows_v):
    wid = lax.axis_index("s") * NC + lax.axis_index("c")
    base = eoff + wid * CH2
    fb = pl.multiple_of((base // 8) * 8, 8)
    shift = base - fb
    lanes = jnp.arange(16, dtype=jnp.int32)
    idx_v[pl.ds(CH2 + 4, 16)] = jnp.zeros((16,), jnp.int32)
    pltpu.sync_copy(src_hbm.at[pl.ds(fb, CH2 + 4)], idx_v.at[pl.ds(0, CH2 + 4)])

    for half, tab in ((0, ha_hbm), (1, hb_hbm)):
        pltpu.sync_copy(tab, htab)

        @plsc.parallel_loop(0, NG2, unroll=8)
        def body(k, half=half):
            sv = plsc.load_gather(idx_v, [shift + k * 16 + lanes])
            addr = sv * 8
            eids = (k * 16 + lanes) * 16 + half * 8
            for w in range(8):
                vals = plsc.load_gather(htab, [addr + w])
                plsc.store_scatter(rows_v, [eids + w], vals)

    pltpu.sync_copy(rows_v.at[pl.ds(0, CH2 * 16)],
                    out_hbm.at[pl.ds((wid * CH2) * 16, CH2 * 16)])

  return _sc_gather


_sc_gather_1 = _make_gather(0)
_sc_gather_2 = _make_gather(ES)


APAD = 640
NPAIR = CHUNK // 2


def _make_scatter(eoff, with_deg):
  n_out = 3 if with_deg else 2

  @functools.partial(
      pl.kernel,
      mesh=_mesh,
      out_type=[jax.ShapeDtypeStruct((NC, APAD, 128), jnp.float32)] * n_out,
      compiler_params=_sc_params,
      scratch_types=[
          pltpu.VMEM((CH2 + 4,), jnp.int32),
          pltpu.VMEM((CH2 * 8,), jnp.float32),
          pltpu.VMEM((APAD, 128), jnp.float32),
          pltpu.VMEM((APAD,), jnp.int32),
          pltpu.VMEM_SHARED((APAD, 128), jnp.float32),
      ],
  )
  def _sc_scatter(msg_hbm, dst_hbm, zero_hbm, *rest):
    if with_deg:
        outa_hbm, outb_hbm, outd_hbm = rest[:3]
    else:
        outa_hbm, outb_hbm = rest[:2]
    idx_v, vals_v, acc_v, iota_v, accum_sh = rest[n_out:]
    c = lax.axis_index("c")
    s = lax.axis_index("s")
    wid = s * NC + c
    base = wid * CH2
    fb = pl.multiple_of(((eoff + base) // 8) * 8, 8)
    shift = (eoff + base) - fb
    lanes = jnp.arange(16, dtype=jnp.int32)
    lo = lanes < 8
    hi = lanes >= 8
    zero16 = jnp.zeros((16,), jnp.float32)

    pltpu.sync_copy(dst_hbm.at[pl.ds(fb, CH2 + 4)], idx_v)

    def iota_fill(r, carry):
        iota_v[pl.ds(r * 16, 16)] = r * 16 + lanes
        return carry

    lax.fori_loop(0, APAD // 16, iota_fill, 0)

    def one_pass(out_hbm, body_of_pass):
        @plsc.parallel_loop(0, APAD, unroll=8)
        def zfill(r):
            for cc in range(8):
                acc_v[r, pl.ds(cc * 16, 16)] = zero16

        pltpu.sync_copy(zero_hbm.at[pl.ds(s * 40, 40)],
                        accum_sh.at[pl.ds(s * 40, 40)])
        body_of_pass()
        plsc.subcore_barrier()
        pltpu.sync_copy(acc_v, accum_sh.at[iota_v], add=True)
        plsc.subcore_barrier()
        pltpu.sync_copy(accum_sh.at[pl.ds(s * 40, 40)],
                        out_hbm.at[c, pl.ds(s * 40, 40)])
        plsc.subcore_barrier()

    for half, out_hbm in ((0, outa_hbm), (1, outb_hbm)):
        def value_pass(half=half):
            for sub in range(2):
                pltpu.sync_copy(
                    msg_hbm.at[pl.ds((base + sub * 1250) * 16, 1250 * 16)],
                    vals_v.at[pl.ds(0, 1250 * 16)])
                e0 = sub * 1250

                @plsc.parallel_loop(0, 625, unroll=4)
                def pair(k, e0=e0, half=half):
                    dpair = plsc.load_gather(
                        idx_v, [shift + e0 + 2 * k + (lanes >> 3)])
                    a = dpair * 8 + (lanes & 7)
                    vals = plsc.load_gather(
                        vals_v,
                        [k * 32 + (lanes >> 3) * 16 + half * 8 + (lanes & 7)])
                    plsc.addupdate_scatter(acc_v, [a >> 7, a & 127], vals,
                                           mask=lo)
                    plsc.addupdate_scatter(acc_v, [a >> 7, a & 127], vals,
                                           mask=hi)

        one_pass(out_hbm, value_pass)

    if with_deg:
        def ones_pass():
            ones16 = jnp.ones((16,), jnp.float32)

            @plsc.parallel_loop(0, CH2 // 2, unroll=4)
            def pair(k):
                dpair = plsc.load_gather(idx_v, [shift + 2 * k + (lanes >> 3)])
                a = dpair * 8 + (lanes & 7)
                plsc.addupdate_scatter(acc_v, [a >> 7, a & 127], ones16,
                                       mask=lo)
                plsc.addupdate_scatter(acc_v, [a >> 7, a & 127], ones16,
                                       mask=hi)

        one_pass(outd_hbm, ones_pass)

  return _sc_scatter


_sc_scatter_1d = _make_scatter(0, True)
_sc_scatter_2d = _make_scatter(ES, True)
_sc_scatter_1n = _make_scatter(0, False)
_sc_scatter_2n = _make_scatter(ES, False)


def _mlp_body(x_ref, w_ref, b_ref, qa_ref, qb_ref, oh_ref, oa_ref, ob_ref):
    y = jnp.dot(x_ref[...], w_ref[...], preferred_element_type=jnp.float32)
    hn = jnp.maximum(y + b_ref[...], 0.0)
    oh_ref[...] = hn
    oa_ref[...] = jnp.dot(hn, qa_ref[...], preferred_element_type=jnp.float32)
    ob_ref[...] = jnp.dot(hn, qb_ref[...], preferred_element_type=jnp.float32)


def _tc_input_mlp(x16, W16, b16, QA, QB):
    return pl.pallas_call(
        _mlp_body,
        out_shape=[jax.ShapeDtypeStruct((NR, 256), jnp.float32),
                   jax.ShapeDtypeStruct((NR, 128), jnp.float32),
                   jax.ShapeDtypeStruct((NR, 128), jnp.float32)],
    )(x16, W16, b16, QA, QB)


def _msg_body(ea_ref, g_ref, a_ref, b1_ref, m_ref, b2_ref, o_ref):
    ea = ea_ref[...]
    g = g_ref[...]
    tb = jnp.dot(ea, a_ref[...], preferred_element_type=jnp.float32)
    tb = jnp.maximum(tb + b1_ref[...], 0.0)
    u = jnp.dot(g, m_ref[...], preferred_element_type=jnp.float32)
    prod = tb * u
    acc = jnp.dot(g, b2_ref[...], preferred_element_type=jnp.float32)
    for b in range(8):
        acc = acc + prod[:, b * 128:(b + 1) * 128]
    o_ref[...] = acc


def _tc_msg(EA8, G8, Astack, b1stack, Mbig, B2big):
    BLK = 400
    rows = G8.shape[0]
    return pl.pallas_call(
        _msg_body,
        grid=(rows // BLK,),
        in_specs=[
            pl.BlockSpec((BLK, 32), lambda i: (i, 0)),
            pl.BlockSpec((BLK, 128), lambda i: (i, 0)),
            pl.BlockSpec((32, 1024), lambda i: (0, 0)),
            pl.BlockSpec((1, 1024), lambda i: (0, 0)),
            pl.BlockSpec((128, 1024), lambda i: (0, 0)),
            pl.BlockSpec((128, 128), lambda i: (0, 0)),
        ],
        out_specs=pl.BlockSpec((BLK, 128), lambda i: (i, 0)),
        out_shape=jax.ShapeDtypeStruct((rows, 128), jnp.float32),
    )(EA8, G8, Astack, b1stack, Mbig, B2big)


def _node_body(h_ref, pa0, pa1, pa2, pa3, pb0, pb1, pb2, pb3,
               da0, da1, da2, da3, wr_ref, br_ref, gm_ref,
               bt_ref, pam_ref, pbm_ref, qa_ref, qb_ref,
               oh_ref, oa_ref, ob_ref):
    f32 = jnp.float32
    h = h_ref[...]
    pam = pam_ref[...]
    pbm = pbm_ref[...]
    agg_a = pa0[...] + pa1[...] + pa2[...] + pa3[...]
    agg_b = pb0[...] + pb1[...] + pb2[...] + pb3[...]
    agg = (jnp.dot(agg_a, pam, preferred_element_type=f32)
           + jnp.dot(agg_b, pbm, preferred_element_type=f32))
    d_a = da0[...] + da1[...] + da2[...] + da3[...]
    deg = jnp.dot(d_a, pam + pbm, preferred_element_type=f32)
    deg = jnp.maximum(deg, 1.0)
    u = jnp.dot(h, wr_ref[...], preferred_element_type=f32)
    u = u + br_ref[...] + agg / deg
    u = u * gm_ref[...] + bt_ref[...]
    hn = jnp.maximum(u, 0.0) + h
    oh_ref[...] = hn
    oa_ref[...] = jnp.dot(hn, qa_ref[...], preferred_element_type=f32)
    ob_ref[...] = jnp.dot(hn, qb_ref[...], preferred_element_type=f32)


def _tc_node(h16, pas, pbs, das, Wr16, br16, gm16, bt16, PA, PB, QA, QB):
    return pl.pallas_call(
        _node_body,
        out_shape=[jax.ShapeDtypeStruct((NR, 256), jnp.float32),
                   jax.ShapeDtypeStruct((NR, 128), jnp.float32),
                   jax.ShapeDtypeStruct((NR, 128), jnp.float32)],
    )(h16, *pas, *pbs, *das, Wr16, br16, gm16, bt16, PA, PB, QA, QB)


def kernel(x, edge_index, edge_attr, W_in, b_in,
           W1_0, b1_0, W2_0, b2_0, Wr_0, br_0, gamma_0, beta_0,
           W1_1, b1_1, W2_1, b2_1, Wr_1, br_1, gamma_1, beta_1,
           W1_2, b1_2, W2_2, b2_2, Wr_2, br_2, gamma_2, beta_2):
    f32 = jnp.float32
    src = edge_index[0]
    dst = edge_index[1]
    zeros_p = jnp.zeros((APAD, 128), f32)
    rs = 1.0 / jnp.sqrt(jnp.asarray(1.0 + EPS, f32))

    eye8 = jnp.eye(8, dtype=f32)
    eye16 = jnp.eye(16, dtype=f32)
    PA = jnp.einsum('ji,cd->cjdi', jnp.eye(8, 16, dtype=f32),
                    eye16).reshape(128, 256)
    PB = jnp.einsum('ji,cd->cjdi', jnp.eye(8, 16, k=8, dtype=f32),
                    eye16).reshape(128, 256)
    QA = jnp.einsum('ij,cd->cidj', jnp.eye(16, 8, dtype=f32),
                    eye16).reshape(256, 128)
    QB = jnp.einsum('ij,cd->cidj', jnp.eye(16, 8, k=-8, dtype=f32),
                    eye16).reshape(256, 128)

    W16 = jnp.einsum('do,ce->cdeo', W_in, eye16).reshape(16 * 128, 256)
    b16 = jnp.tile(b_in, 16).reshape(1, 256)

    x16 = x.reshape(NR, 16 * 128)
    EA8_1 = edge_attr[:ES].reshape(ES // 8, 32)
    EA8_2 = edge_attr[ES:].reshape(ES // 8, 32)

    h16, hap, hbp = _tc_input_mlp(x16, W16, b16, QA, QB)
    das = None

    layers = [
        (W1_0, b1_0, W2_0, b2_0, Wr_0, br_0, gamma_0, beta_0),
        (W1_1, b1_1, W2_1, b2_1, Wr_1, br_1, gamma_1, beta_1),
        (W1_2, b1_2, W2_2, b2_2, Wr_2, br_2, gamma_2, beta_2),
    ]
    for (W1, b1, W2, b2, Wr, br, gm, bt) in layers:
        Astack = (W1[None, :, :, None, None] * eye8[:, None, None, :, None])
        Astack = jnp.broadcast_to(Astack, (8, 4, 8, 8, 16)).reshape(32, 1024)
        b1stack = jnp.repeat(b1, 128).reshape(1, 1024)
        W2r = W2.reshape(8, H, H)
        Mbig = jnp.einsum('bio,cd->cibdo', W2r, eye8).reshape(128, 1024)
        B2big = jnp.einsum('io,cd->cido', b2.reshape(H, H),
                           eye8).reshape(128, 128)
        Wr16 = jnp.einsum('io,cd->cido', Wr, eye16).reshape(256, 256)
        br16 = jnp.tile(br, 16).reshape(1, 256)
        gm16 = jnp.tile(gm * rs, 16).reshape(1, 256)
        bt16 = jnp.tile(bt, 16).reshape(1, 256)

        ha_l = hap.reshape(N * 8)
        hb_l = hbp.reshape(N * 8)
        g8_1 = _sc_gather_1(ha_l, hb_l, src)
        g8_2 = _sc_gather_2(ha_l, hb_l, src)
        msg1 = _tc_msg(EA8_1, g8_1.reshape(ES // 8, 128),
                       Astack, b1stack, Mbig, B2big)
        msg2 = _tc_msg(EA8_2, g8_2.reshape(ES // 8, 128),
                       Astack, b1stack, Mbig, B2big)
        if das is None:
            pa1_, pb1_, pd1 = _sc_scatter_1d(msg1.reshape(ES * 16), dst,
                                             zeros_p)
            pa2_, pb2_, pd2 = _sc_scatter_2d(msg2.reshape(ES * 16), dst,
                                             zeros_p)
            das = [pd1[0, :NR], pd1[1, :NR], pd2[0, :NR], pd2[1, :NR]]
        else:
            pa1_, pb1_ = _sc_scatter_1n(msg1.reshape(ES * 16), dst, zeros_p)
            pa2_, pb2_ = _sc_scatter_2n(msg2.reshape(ES * 16), dst, zeros_p)
        pas = [pa1_[0, :NR], pa1_[1, :NR], pa2_[0, :NR], pa2_[1, :NR]]
        pbs = [pb1_[0, :NR], pb1_[1, :NR], pb2_[0, :NR], pb2_[1, :NR]]
        h16, hap, hbp = _tc_node(h16, pas, pbs, das,
                                 Wr16, br16, gm16, bt16, PA, PB, QA, QB)
    return h16.reshape(N, H)


# speedup vs baseline: 5.9269x; 1.0241x over previous
import functools

import jax
import jax.numpy as jnp
from jax import lax
from jax.experimental import pallas as pl
from jax.experimental.pallas import tpu as pltpu
from jax.experimental.pallas import tpu_sc as plsc

N = 10000
E = 160000
H = 16
EPS = 1e-5

NC = 2
NS = 16
NW = NC * NS
CHUNK = E // NW
NR = N // 16
ER = E // 8

_mesh = plsc.VectorSubcoreMesh(core_axis_name="c", subcore_axis_name="s")
_sc_params = pltpu.CompilerParams(needs_layout_passes=False)


ES = E // 2
CH2 = ES // NW
NG2 = 157
RW2 = NG2 * 16 * 16


def _make_gather(eoff):
  @functools.partial(
      pl.kernel,
      mesh=_mesh,
      out_type=jax.ShapeDtypeStruct((ES * 16,), jnp.float32),
      compiler_params=_sc_params,
      scratch_types=[
          pltpu.VMEM((CH2 + 20,), jnp.int32),
          pltpu.VMEM((N * 8,), jnp.float32),
          pltpu.VMEM((RW2,), jnp.float32),
      ],
  )
  def _sc_gather(ha_hbm, hb_hbm, src_hbm, out_hbm, idx_v, htab, rows_v):
    wid = lax.axis_index("s") * NC + lax.axis_index("c")
    base = eoff + wid * CH2
    fb = pl.multiple_of((base // 8) * 8, 8)
    shift = base - fb
    lanes = jnp.arange(16, dtype=jnp.int32)
    idx_v[pl.ds(CH2 + 4, 16)] = jnp.zeros((16,), jnp.int32)
    pltpu.sync_copy(src_hbm.at[pl.ds(fb, CH2 + 4)], idx_v.at[pl.ds(0, CH2 + 4)])

    for half, tab in ((0, ha_hbm), (1, hb_hbm)):
        pltpu.sync_copy(tab, htab)

        @plsc.parallel_loop(0, NG2, unroll=8)
        def body(k, half=half):
            sv = plsc.load_gather(idx_v, [shift + k * 16 + lanes])
            addr = sv * 8
            eids = (k * 16 + lanes) * 16 + half * 8
            for w in range(8):
                vals = plsc.load_gather(htab, [addr + w])
                plsc.store_scatter(rows_v, [eids + w], vals)

    pltpu.sync_copy(rows_v.at[pl.ds(0, CH2 * 16)],
                    out_hbm.at[pl.ds((wid * CH2) * 16, CH2 * 16)])

  return _sc_gather


_sc_gather_1 = _make_gather(0)
_sc_gather_2 = _make_gather(ES)


APAD = 640
NPAIR = CHUNK // 2


def _make_scatter(eoff, with_deg):
  n_out = 3 if with_deg else 2

  @functools.partial(
      pl.kernel,
      mesh=_mesh,
      out_type=[jax.ShapeDtypeStruct((NC, APAD, 128), jnp.float32)] * n_out,
      compiler_params=_sc_params,
      scratch_types=[
          pltpu.VMEM((CH2 + 4,), jnp.int32),
          pltpu.VMEM((CH2 * 8,), jnp.float32),
          pltpu.VMEM((APAD, 128), jnp.float32),
          pltpu.VMEM((APAD,), jnp.int32),
          pltpu.VMEM_SHARED((APAD, 128), jnp.float32),
      ],
  )
  def _sc_scatter(msg_hbm, dst_hbm, zero_hbm, *rest):
    if with_deg:
        outa_hbm, outb_hbm, outd_hbm = rest[:3]
    else:
        outa_hbm, outb_hbm = rest[:2]
    idx_v, vals_v, acc_v, iota_v, accum_sh = rest[n_out:]
    c = lax.axis_index("c")
    s = lax.axis_index("s")
    wid = s * NC + c
    base = wid * CH2
    fb = pl.multiple_of(((eoff + base) // 8) * 8, 8)
    shift = (eoff + base) - fb
    lanes = jnp.arange(16, dtype=jnp.int32)
    lo = lanes < 8
    hi = lanes >= 8
    zero16 = jnp.zeros((16,), jnp.float32)

    pltpu.sync_copy(dst_hbm.at[pl.ds(fb, CH2 + 4)], idx_v)

    def iota_fill(r, carry):
        iota_v[pl.ds(r * 16, 16)] = r * 16 + lanes
        return carry

    lax.fori_loop(0, APAD // 16, iota_fill, 0)

    def one_pass(out_hbm, body_of_pass):
        @plsc.parallel_loop(0, APAD, unroll=8)
        def zfill(r):
            for cc in range(8):
                acc_v[r, pl.ds(cc * 16, 16)] = zero16

        pltpu.sync_copy(zero_hbm.at[pl.ds(s * 40, 40)],
                        accum_sh.at[pl.ds(s * 40, 40)])
        body_of_pass()
        plsc.subcore_barrier()
        pltpu.sync_copy(acc_v, accum_sh.at[iota_v], add=True)
        plsc.subcore_barrier()
        pltpu.sync_copy(accum_sh.at[pl.ds(s * 40, 40)],
                        out_hbm.at[c, pl.ds(s * 40, 40)])
        plsc.subcore_barrier()

    for half, out_hbm in ((0, outa_hbm), (1, outb_hbm)):
        def value_pass(half=half):
            for sub in range(2):
                pltpu.sync_copy(
                    msg_hbm.at[pl.ds((base + sub * 1250) * 16, 1250 * 16)],
                    vals_v.at[pl.ds(0, 1250 * 16)])
                e0 = sub * 1250

                @plsc.parallel_loop(0, 625, unroll=4)
                def pair(k, e0=e0, half=half):
                    dpair = plsc.load_gather(
                        idx_v, [shift + e0 + 2 * k + (lanes >> 3)])
                    a = dpair * 8 + (lanes & 7)
                    vals = plsc.load_gather(
                        vals_v,
                        [k * 32 + (lanes >> 3) * 16 + half * 8 + (lanes & 7)])
                    plsc.addupdate_scatter(acc_v, [a >> 7, a & 127], vals,
                                           mask=lo)
                    plsc.addupdate_scatter(acc_v, [a >> 7, a & 127], vals,
                                           mask=hi)

        one_pass(out_hbm, value_pass)

    if with_deg:
        def ones_pass():
            ones16 = jnp.ones((16,), jnp.float32)

            @plsc.parallel_loop(0, CH2 // 2, unroll=4)
            def pair(k):
                dpair = plsc.load_gather(idx_v, [shift + 2 * k + (lanes >> 3)])
                a = dpair * 8 + (lanes & 7)
                plsc.addupdate_scatter(acc_v, [a >> 7, a & 127], ones16,
                                       mask=lo)
                plsc.addupdate_scatter(acc_v, [a >> 7, a & 127], ones16,
                                       mask=hi)

        one_pass(outd_hbm, ones_pass)

  return _sc_scatter


_sc_scatter_1d = _make_scatter(0, True)
_sc_scatter_2d = _make_scatter(ES, True)
_sc_scatter_1n = _make_scatter(0, False)
_sc_scatter_2n = _make_scatter(ES, False)


def _mlp_body(x_ref, w_ref, b_ref, qa_ref, qb_ref, oh_ref, oa_ref, ob_ref):
    y = jnp.dot(x_ref[...], w_ref[...], preferred_element_type=jnp.float32)
    hn = jnp.maximum(y + b_ref[...], 0.0)
    oh_ref[...] = hn
    oa_ref[...] = jnp.dot(hn, qa_ref[...], preferred_element_type=jnp.float32)
    ob_ref[...] = jnp.dot(hn, qb_ref[...], preferred_element_type=jnp.float32)


def _tc_input_mlp(x16, W16, b16, QA, QB):
    return pl.pallas_call(
        _mlp_body,
        out_shape=[jax.ShapeDtypeStruct((NR, 256), jnp.float32),
                   jax.ShapeDtypeStruct((NR, 128), jnp.float32),
                   jax.ShapeDtypeStruct((NR, 128), jnp.float32)],
    )(x16, W16, b16, QA, QB)


def _msg_body(ea_ref, g_ref, a_ref, b1_ref, m_ref, b2_ref, o_ref):
    ea = ea_ref[...]
    g = g_ref[...]
    tb = jnp.dot(ea, a_ref[...], preferred_element_type=jnp.float32)
    tb = jnp.maximum(tb + b1_ref[...], 0.0)
    u = jnp.dot(g, m_ref[...], preferred_element_type=jnp.float32)
    prod = tb * u
    acc = jnp.dot(g, b2_ref[...], preferred_element_type=jnp.float32)
    for b in range(8):
        acc = acc + prod[:, b * 128:(b + 1) * 128]
    o_ref[...] = acc


def _tc_msg(EA8, G8, Astack, b1stack, Mbig, B2big):
    BLK = 400
    rows = G8.shape[0]
    return pl.pallas_call(
        _msg_body,
        grid=(rows // BLK,),
        in_specs=[
            pl.BlockSpec((BLK, 32), lambda i: (i, 0)),
            pl.BlockSpec((BLK, 128), lambda i: (i, 0)),
            pl.BlockSpec((32, 1024), lambda i: (0, 0)),
            pl.BlockSpec((1, 1024), lambda i: (0, 0)),
            pl.BlockSpec((128, 1024), lambda i: (0, 0)),
            pl.BlockSpec((128, 128), lambda i: (0, 0)),
        ],
        out_specs=pl.BlockSpec((BLK, 128), lambda i: (i, 0)),
        out_shape=jax.ShapeDtypeStruct((rows, 128), jnp.float32),
    )(EA8, G8, Astack, b1stack, Mbig, B2big)


def _node_body(h_ref, pa1_, pa2_, pb1_, pb2_, da1_, da2_,
               wr_ref, br_ref, gm_ref,
               bt_ref, pam_ref, pbm_ref, qa_ref, qb_ref,
               oh_ref, oa_ref, ob_ref):
    f32 = jnp.float32

    def tot(r):
        v = r[...]
        return v[0, :NR] + v[1, :NR]

    h = h_ref[...]
    pam = pam_ref[...]
    pbm = pbm_ref[...]
    agg_a = tot(pa1_) + tot(pa2_)
    agg_b = tot(pb1_) + tot(pb2_)
    agg = (jnp.dot(agg_a, pam, preferred_element_type=f32)
           + jnp.dot(agg_b, pbm, preferred_element_type=f32))
    d_a = tot(da1_) + tot(da2_)
    deg = jnp.dot(d_a, pam + pbm, preferred_element_type=f32)
    deg = jnp.maximum(deg, 1.0)
    u = jnp.dot(h, wr_ref[...], preferred_element_type=f32)
    u = u + br_ref[...] + agg / deg
    u = u * gm_ref[...] + bt_ref[...]
    hn = jnp.maximum(u, 0.0) + h
    oh_ref[...] = hn
    oa_ref[...] = jnp.dot(hn, qa_ref[...], preferred_element_type=f32)
    ob_ref[...] = jnp.dot(hn, qb_ref[...], preferred_element_type=f32)


def _tc_node(h16, pas, pbs, das, Wr16, br16, gm16, bt16, PA, PB, QA, QB):
    return pl.pallas_call(
        _node_body,
        out_shape=[jax.ShapeDtypeStruct((NR, 256), jnp.float32),
                   jax.ShapeDtypeStruct((NR, 128), jnp.float32),
                   jax.ShapeDtypeStruct((NR, 128), jnp.float32)],
    )(h16, *pas, *pbs, *das, Wr16, br16, gm16, bt16, PA, PB, QA, QB)


def kernel(x, edge_index, edge_attr, W_in, b_in,
           W1_0, b1_0, W2_0, b2_0, Wr_0, br_0, gamma_0, beta_0,
           W1_1, b1_1, W2_1, b2_1, Wr_1, br_1, gamma_1, beta_1,
           W1_2, b1_2, W2_2, b2_2, Wr_2, br_2, gamma_2, beta_2):
    f32 = jnp.float32
    src = edge_index[0]
    dst = edge_index[1]
    zeros_p = jnp.zeros((APAD, 128), f32)
    rs = 1.0 / jnp.sqrt(jnp.asarray(1.0 + EPS, f32))

    eye8 = jnp.eye(8, dtype=f32)
    eye16 = jnp.eye(16, dtype=f32)
    PA = jnp.einsum('ji,cd->cjdi', jnp.eye(8, 16, dtype=f32),
                    eye16).reshape(128, 256)
    PB = jnp.einsum('ji,cd->cjdi', jnp.eye(8, 16, k=8, dtype=f32),
                    eye16).reshape(128, 256)
    QA = jnp.einsum('ij,cd->cidj', jnp.eye(16, 8, dtype=f32),
                    eye16).reshape(256, 128)
    QB = jnp.einsum('ij,cd->cidj', jnp.eye(16, 8, k=-8, dtype=f32),
                    eye16).reshape(256, 128)

    W16 = jnp.einsum('do,ce->cdeo', W_in, eye16).reshape(16 * 128, 256)
    b16 = jnp.tile(b_in, 16).reshape(1, 256)

    x16 = x.reshape(NR, 16 * 128)
    EA8_1 = edge_attr[:ES].reshape(ES // 8, 32)
    EA8_2 = edge_attr[ES:].reshape(ES // 8, 32)

    h16, hap, hbp = _tc_input_mlp(x16, W16, b16, QA, QB)
    das = None

    layers = [
        (W1_0, b1_0, W2_0, b2_0, Wr_0, br_0, gamma_0, beta_0),
        (W1_1, b1_1, W2_1, b2_1, Wr_1, br_1, gamma_1, beta_1),
        (W1_2, b1_2, W2_2, b2_2, Wr_2, br_2, gamma_2, beta_2),
    ]
    for (W1, b1, W2, b2, Wr, br, gm, bt) in layers:
        Astack = (W1[None, :, :, None, None] * eye8[:, None, None, :, None])
        Astack = jnp.broadcast_to(Astack, (8, 4, 8, 8, 16)).reshape(32, 1024)
        b1stack = jnp.repeat(b1, 128).reshape(1, 1024)
        W2r = W2.reshape(8, H, H)
        Mbig = jnp.einsum('bio,cd->cibdo', W2r, eye8).reshape(128, 1024)
        B2big = jnp.einsum('io,cd->cido', b2.reshape(H, H),
                           eye8).reshape(128, 128)
        Wr16 = jnp.einsum('io,cd->cido', Wr, eye16).reshape(256, 256)
        br16 = jnp.tile(br, 16).reshape(1, 256)
        gm16 = jnp.tile(gm * rs, 16).reshape(1, 256)
        bt16 = jnp.tile(bt, 16).reshape(1, 256)

        ha_l = hap.reshape(N * 8)
        hb_l = hbp.reshape(N * 8)
        g8_1 = _sc_gather_1(ha_l, hb_l, src)
        g8_2 = _sc_gather_2(ha_l, hb_l, src)
        msg1 = _tc_msg(EA8_1, g8_1.reshape(ES // 8, 128),
                       Astack, b1stack, Mbig, B2big)
        msg2 = _tc_msg(EA8_2, g8_2.reshape(ES // 8, 128),
                       Astack, b1stack, Mbig, B2big)
        if das is None:
            pa1_, pb1_, pd1 = _sc_scatter_1d(msg1.reshape(ES * 16), dst,
                                             zeros_p)
            pa2_, pb2_, pd2 = _sc_scatter_2d(msg2.reshape(ES * 16), dst,
                                             zeros_p)
            das = [pd1, pd2]
        else:
            pa1_, pb1_ = _sc_scatter_1n(msg1.reshape(ES * 16), dst, zeros_p)
            pa2_, pb2_ = _sc_scatter_2n(msg2.reshape(ES * 16), dst, zeros_p)
        h16, hap, hbp = _tc_node(h16, [pa1_, pa2_], [pb1_, pb2_], das,
                                 Wr16, br16, gm16, bt16, PA, PB, QA, QB)
    return h16.reshape(N, H)
